# baseline (speedup 1.0000x reference)
import functools

import numpy as np

import jax
import jax.numpy as jnp
from jax import lax
from jax.experimental import pallas as pl
from jax.experimental.pallas import tpu as pltpu
from jax.experimental.pallas import tpu_sc as plsc

B, N, D = 2, 256, 256
HIDDEN = 256
EDGE_NUM = 16
SIMILAR_EDGE = 8
UCHUNK = 128


def _np_threefry_uniform(seed, shape):
    def rotl(v, r):
        return ((v << np.uint32(r)) | (v >> np.uint32(32 - r))).astype(np.uint32)

    n = int(np.prod(shape))
    x0 = np.zeros(n, dtype=np.uint32)
    x1 = np.arange(n, dtype=np.uint32)
    ks0, ks1 = np.uint32(0), np.uint32(seed)
    ks2 = np.uint32(ks0 ^ ks1 ^ np.uint32(0x1BD11BDA))
    x0 = (x0 + ks0).astype(np.uint32)
    x1 = (x1 + ks1).astype(np.uint32)
    rot = ((13, 15, 26, 6), (17, 29, 16, 24))
    ks = (ks1, ks2, ks0, ks1, ks2, ks0)
    for i in range(5):
        for r in rot[i % 2]:
            x0 = (x0 + x1).astype(np.uint32)
            x1 = (rotl(x1, r) ^ x0).astype(np.uint32)
        x0 = (x0 + ks[i]).astype(np.uint32)
        x1 = (x1 + ks[i + 1] + np.uint32(i + 1)).astype(np.uint32)
    bits = (x0 ^ x1).astype(np.uint32)
    fl = ((bits >> np.uint32(9)) | np.uint32(0x3F800000)).view(np.float32) - np.float32(1.0)
    return fl.reshape(shape)


_UCONST = _np_threefry_uniform(1, (B, N, N))

NEGINF = float(np.finfo(np.float32).min)


def _tc_body(x_ref, adj_ref, w1_ref, b1_ref, w2_ref, u_ref, yt_ref, adjt_ref,
             a_s, bv_s, logit_s):
    x = x_ref[0]
    w1 = w1_ref[...]
    dn = (((1,), (1,)), ((), ()))
    a_s[...] = lax.dot_general(x, w1[:, :D], dn, preferred_element_type=jnp.float32) + b1_ref[...]
    bv_s[...] = lax.dot_general(x, w1[:, D:], dn, preferred_element_type=jnp.float32)

    w2p = jnp.broadcast_to(w2_ref[...], (8, HIDDEN))

    def chunk(i, _):
        a_blk = a_s[pl.ds(i * UCHUNK, UCHUNK), :]
        t = jnp.maximum(a_blk[:, None, :] + bv_s[...][None, :, :], 0.0)
        lg = lax.dot_general(t.reshape(UCHUNK * N, HIDDEN), w2p, dn,
                             preferred_element_type=jnp.float32)
        logit_s[pl.ds(i * UCHUNK, UCHUNK), :] = lg[:, 0:1].reshape(UCHUNK, N)
        return 0

    lax.fori_loop(0, N // UCHUNK, chunk, 0)

    adj = adj_ref[0]
    edge = adj != 0.0
    u = u_ref[0]
    g = -jnp.log(-jnp.log(jnp.clip(u, 1e-10, 1.0 - 1e-10)))
    y = jnp.where(edge, logit_s[...] + g, NEGINF)
    yt_ref[0] = y.T
    adjt_ref[0] = adj.T


@jax.jit
def _tc_y(X, Adj, W1, b1, W2):
    U = jnp.asarray(_UCONST)
    return pl.pallas_call(
        _tc_body,
        grid=(B,),
        in_specs=[
            pl.BlockSpec((1, N, D), lambda b: (b, 0, 0)),
            pl.BlockSpec((1, N, N), lambda b: (b, 0, 0)),
            pl.BlockSpec((HIDDEN, 2 * D), lambda b: (0, 0)),
            pl.BlockSpec((1, HIDDEN), lambda b: (0, 0)),
            pl.BlockSpec((1, HIDDEN), lambda b: (0, 0)),
            pl.BlockSpec((1, N, N), lambda b: (b, 0, 0)),
        ],
        out_specs=[
            pl.BlockSpec((1, N, N), lambda b: (b, 0, 0)),
            pl.BlockSpec((1, N, N), lambda b: (b, 0, 0)),
        ],
        out_shape=[
            jax.ShapeDtypeStruct((B, N, N), jnp.float32),
            jax.ShapeDtypeStruct((B, N, N), jnp.float32),
        ],
        scratch_shapes=[
            pltpu.VMEM((N, HIDDEN), jnp.float32),
            pltpu.VMEM((N, HIDDEN), jnp.float32),
            pltpu.VMEM((N, N), jnp.float32),
        ],
    )(X, Adj, W1, b1.reshape(1, HIDDEN), W2, U)


ROWS_PER_TILE = 16
TILES_PER_BATCH = N // ROWS_PER_TILE


def _make_sc_mask():
    info = plsc.get_sparse_core_info()
    nc = info.num_cores
    mesh = plsc.VectorSubcoreMesh(core_axis_name="c", subcore_axis_name="s")

    TE = N * ROWS_PER_TILE

    @functools.partial(
        pl.kernel, mesh=mesh,
        out_type=jax.ShapeDtypeStruct((B * N * N,), jnp.float32),
        scratch_types=[
            pltpu.VMEM((TE,), jnp.float32),
            pltpu.VMEM((TE,), jnp.float32),
            pltpu.VMEM((TE,), jnp.float32),
        ],
    )
    def sc_mask(ys_hbm, as_hbm, outs_hbm, yt_v, at_v, ot_v):
        wid = lax.axis_index("s") * nc + lax.axis_index("c")
        base = wid * TE
        pltpu.sync_copy(ys_hbm.at[pl.ds(base, TE)], yt_v)
        pltpu.sync_copy(as_hbm.at[pl.ds(base, TE)], at_v)

        neg_inf = jnp.full((ROWS_PER_TILE,), -jnp.inf, jnp.float32)

        def round_(t, _):
            def mx(j, m):
                return jnp.maximum(m, yt_v[pl.ds(j * 16, 16)])
            m = lax.fori_loop(0, N, mx, neg_inf)

            def rm(j, _):
                v = yt_v[pl.ds(j * 16, 16)]
                yt_v[pl.ds(j * 16, 16)] = jnp.where(v == m, -jnp.inf, v)
                return 0
            lax.fori_loop(0, N, rm, 0)
            return 0
        lax.fori_loop(0, EDGE_NUM, round_, 0)

        def cnt(j, c):
            return c + at_v[pl.ds(j * 16, 16)]
        r_tot = lax.fori_loop(0, N, cnt, jnp.zeros((ROWS_PER_TILE,), jnp.float32))

        k8 = jnp.full((ROWS_PER_TILE,), float(SIMILAR_EDGE), jnp.float32)
        ones = jnp.full((ROWS_PER_TILE,), 1.0, jnp.float32)
        zeros = jnp.zeros((ROWS_PER_TILE,), jnp.float32)
        rlt = jnp.where(r_tot < k8, ones, zeros)

        def emit(j, c):
            a = at_v[pl.ds(j * 16, 16)]
            c = c + a
            s16 = jnp.where(yt_v[pl.ds(j * 16, 16)] == -jnp.inf, ones, zeros)
            cle = jnp.where(c <= k8, ones, zeros)
            pos1 = (j + 1).astype(jnp.float32) + zeros
            zle = jnp.where((pos1 - c) <= (k8 - r_tot), ones, zeros)
            sel8 = a * cle + (ones - a) * rlt * zle
            ot_v[pl.ds(j * 16, 16)] = jnp.minimum(sel8 + s16 * a, ones)
            return c
        lax.fori_loop(0, N, emit, jnp.zeros((ROWS_PER_TILE,), jnp.float32))

        pltpu.sync_copy(ot_v, outs_hbm.at[pl.ds(base, TE)])

    return sc_mask


_sc_mask = _make_sc_mask()


@jax.jit
def kernel(X, Adj, W1, b1, W2, b2):
    del b2
    yt, adjt = _tc_y(X, Adj, W1, b1, W2)
    ys = yt.reshape(B, N, TILES_PER_BATCH, ROWS_PER_TILE).transpose(0, 2, 1, 3).reshape(-1)
    ads = adjt.reshape(B, N, TILES_PER_BATCH, ROWS_PER_TILE).transpose(0, 2, 1, 3).reshape(-1)
    outs = _sc_mask(ys, ads)
    return (outs.reshape(B, TILES_PER_BATCH, N, ROWS_PER_TILE)
            .transpose(0, 1, 3, 2).reshape(B, N, N))


# speedup vs baseline: 1.2560x; 1.2560x over previous
import functools

import numpy as np

import jax
import jax.numpy as jnp
from jax import lax
from jax.experimental import pallas as pl
from jax.experimental.pallas import tpu as pltpu
from jax.experimental.pallas import tpu_sc as plsc

B, N, D = 2, 256, 256
HIDDEN = 256
EDGE_NUM = 16
SIMILAR_EDGE = 8
UCHUNK = 128


def _np_threefry_uniform(seed, shape):
    def rotl(v, r):
        return ((v << np.uint32(r)) | (v >> np.uint32(32 - r))).astype(np.uint32)

    n = int(np.prod(shape))
    x0 = np.zeros(n, dtype=np.uint32)
    x1 = np.arange(n, dtype=np.uint32)
    ks0, ks1 = np.uint32(0), np.uint32(seed)
    ks2 = np.uint32(ks0 ^ ks1 ^ np.uint32(0x1BD11BDA))
    x0 = (x0 + ks0).astype(np.uint32)
    x1 = (x1 + ks1).astype(np.uint32)
    rot = ((13, 15, 26, 6), (17, 29, 16, 24))
    ks = (ks1, ks2, ks0, ks1, ks2, ks0)
    for i in range(5):
        for r in rot[i % 2]:
            x0 = (x0 + x1).astype(np.uint32)
            x1 = (rotl(x1, r) ^ x0).astype(np.uint32)
        x0 = (x0 + ks[i]).astype(np.uint32)
        x1 = (x1 + ks[i + 1] + np.uint32(i + 1)).astype(np.uint32)
    bits = (x0 ^ x1).astype(np.uint32)
    fl = ((bits >> np.uint32(9)) | np.uint32(0x3F800000)).view(np.float32) - np.float32(1.0)
    return fl.reshape(shape)


_UCONST = _np_threefry_uniform(1, (B, N, N))

NEGINF = float(np.finfo(np.float32).min)


def _tc_body(x_ref, adj_ref, w1_ref, b1_ref, w2_ref, u_ref, yt_ref, adjt_ref,
             a_s, bv_s, logit_s):
    x = x_ref[0]
    w1 = w1_ref[...]
    dn = (((1,), (1,)), ((), ()))
    a_s[...] = lax.dot_general(x, w1[:, :D], dn, preferred_element_type=jnp.float32) + b1_ref[...]
    bv_s[...] = lax.dot_general(x, w1[:, D:], dn, preferred_element_type=jnp.float32)

    w2p = jnp.broadcast_to(w2_ref[...], (8, HIDDEN))

    def chunk(i, _):
        a_blk = a_s[pl.ds(i * UCHUNK, UCHUNK), :]
        t = jnp.maximum(a_blk[:, None, :] + bv_s[...][None, :, :], 0.0)
        lg = lax.dot_general(t.reshape(UCHUNK * N, HIDDEN), w2p, dn,
                             preferred_element_type=jnp.float32)
        logit_s[pl.ds(i * UCHUNK, UCHUNK), :] = lg[:, 0:1].reshape(UCHUNK, N)
        return 0

    lax.fori_loop(0, N // UCHUNK, chunk, 0)

    adj = adj_ref[0]
    edge = adj != 0.0
    u = u_ref[0]
    g = -jnp.log(-jnp.log(jnp.clip(u, 1e-10, 1.0 - 1e-10)))
    y = jnp.where(edge, logit_s[...] + g, NEGINF)
    yt_ref[0] = y.T
    adjt_ref[0] = adj.T


@jax.jit
def _tc_y(X, Adj, W1, b1, W2):
    U = jnp.asarray(_UCONST)
    return pl.pallas_call(
        _tc_body,
        grid=(B,),
        in_specs=[
            pl.BlockSpec((1, N, D), lambda b: (b, 0, 0)),
            pl.BlockSpec((1, N, N), lambda b: (b, 0, 0)),
            pl.BlockSpec((HIDDEN, 2 * D), lambda b: (0, 0)),
            pl.BlockSpec((1, HIDDEN), lambda b: (0, 0)),
            pl.BlockSpec((1, HIDDEN), lambda b: (0, 0)),
            pl.BlockSpec((1, N, N), lambda b: (b, 0, 0)),
        ],
        out_specs=[
            pl.BlockSpec((1, N, N), lambda b: (b, 0, 0)),
            pl.BlockSpec((1, N, N), lambda b: (b, 0, 0)),
        ],
        out_shape=[
            jax.ShapeDtypeStruct((B, N, N), jnp.float32),
            jax.ShapeDtypeStruct((B, N, N), jnp.float32),
        ],
        scratch_shapes=[
            pltpu.VMEM((N, HIDDEN), jnp.float32),
            pltpu.VMEM((N, HIDDEN), jnp.float32),
            pltpu.VMEM((N, N), jnp.float32),
        ],
    )(X, Adj, W1, b1.reshape(1, HIDDEN), W2, U)


ROWS_PER_TILE = 16
TILES_PER_BATCH = N // ROWS_PER_TILE


def _make_sc_mask():
    info = plsc.get_sparse_core_info()
    nc = info.num_cores
    mesh = plsc.VectorSubcoreMesh(core_axis_name="c", subcore_axis_name="s")

    TE = N * ROWS_PER_TILE

    @functools.partial(
        pl.kernel, mesh=mesh,
        out_type=jax.ShapeDtypeStruct((B * N * N,), jnp.float32),
        scratch_types=[
            pltpu.VMEM((TE,), jnp.float32),
            pltpu.VMEM((TE,), jnp.float32),
            pltpu.VMEM((TE,), jnp.float32),
        ],
    )
    def sc_mask(ys_hbm, as_hbm, outs_hbm, yt_v, at_v, ot_v):
        wid = lax.axis_index("s") * nc + lax.axis_index("c")
        base = wid * TE
        pltpu.sync_copy(ys_hbm.at[pl.ds(base, TE)], yt_v)
        pltpu.sync_copy(as_hbm.at[pl.ds(base, TE)], at_v)

        neg_inf = jnp.full((ROWS_PER_TILE,), -jnp.inf, jnp.float32)
        pos_inf = jnp.full((ROWS_PER_TILE,), jnp.inf, jnp.float32)

        def round_(t, m_cur):
            def mx(j, m):
                v = yt_v[pl.ds(j * 16, 16)]
                return jnp.maximum(m, jnp.where(v < m_cur, v, neg_inf))
            return lax.fori_loop(0, N, mx, neg_inf)
        thresh = lax.fori_loop(0, EDGE_NUM, round_, pos_inf)

        def cnt(j, c):
            return c + at_v[pl.ds(j * 16, 16)]
        r_tot = lax.fori_loop(0, N, cnt, jnp.zeros((ROWS_PER_TILE,), jnp.float32))

        k8 = jnp.full((ROWS_PER_TILE,), float(SIMILAR_EDGE), jnp.float32)
        ones = jnp.full((ROWS_PER_TILE,), 1.0, jnp.float32)
        zeros = jnp.zeros((ROWS_PER_TILE,), jnp.float32)
        rlt = jnp.where(r_tot < k8, ones, zeros)

        def emit(j, c):
            a = at_v[pl.ds(j * 16, 16)]
            c = c + a
            s16 = jnp.where(yt_v[pl.ds(j * 16, 16)] >= thresh, ones, zeros)
            cle = jnp.where(c <= k8, ones, zeros)
            pos1 = (j + 1).astype(jnp.float32) + zeros
            zle = jnp.where((pos1 - c) <= (k8 - r_tot), ones, zeros)
            sel8 = a * cle + (ones - a) * rlt * zle
            ot_v[pl.ds(j * 16, 16)] = jnp.minimum(sel8 + s16 * a, ones)
            return c
        lax.fori_loop(0, N, emit, jnp.zeros((ROWS_PER_TILE,), jnp.float32))

        pltpu.sync_copy(ot_v, outs_hbm.at[pl.ds(base, TE)])

    return sc_mask


_sc_mask = _make_sc_mask()


@jax.jit
def kernel(X, Adj, W1, b1, W2, b2):
    del b2
    yt, adjt = _tc_y(X, Adj, W1, b1, W2)
    ys = yt.reshape(B, N, TILES_PER_BATCH, ROWS_PER_TILE).transpose(0, 2, 1, 3).reshape(-1)
    ads = adjt.reshape(B, N, TILES_PER_BATCH, ROWS_PER_TILE).transpose(0, 2, 1, 3).reshape(-1)
    outs = _sc_mask(ys, ads)
    return (outs.reshape(B, TILES_PER_BATCH, N, ROWS_PER_TILE)
            .transpose(0, 1, 3, 2).reshape(B, N, N))
